# baseline (speedup 1.0000x reference)
import functools

import jax
import jax.numpy as jnp
import numpy as np
from jax import lax
from jax.experimental import pallas as pl
from jax.experimental.pallas import tpu as pltpu
from jax.experimental.pallas import tpu_sc as plsc

_POOL_H = 7
_POOL_W = 7
_SCALE = 0.125
_S = 2

_NPAD = 1024
_NROW = 112
_NCHUNK = 7
_ROWS_PER_BIN = 16
_BLK = 128


def _prep_body(rois_ref, b_ref, idx_ref, w_ref, *, H, W):
    x1 = rois_ref[:, 0:1] * _SCALE
    y1 = rois_ref[:, 1:2] * _SCALE
    x2 = rois_ref[:, 2:3] * _SCALE
    y2 = rois_ref[:, 3:4] * _SCALE
    roi_w = jnp.maximum(x2 - x1, 1.0)
    roi_h = jnp.maximum(y2 - y1, 1.0)
    bin_w = roi_w / _POOL_W
    bin_h = roi_h / _POOL_H

    j = lax.broadcasted_iota(jnp.int32, (_BLK, _NCHUNK * _NROW), 1)
    ph = j // _NROW
    pw = (j % _NROW) // _ROWS_PER_BIN
    s = (j % _ROWS_PER_BIN) // 4
    iy = s // 2
    ix = s % 2
    c = j % 4
    cy = c // 2
    cx = c % 2

    yy = y1 + (ph.astype(jnp.float32)
               + (iy.astype(jnp.float32) + 0.5) / _S) * bin_h
    xx = x1 + (pw.astype(jnp.float32)
               + (ix.astype(jnp.float32) + 0.5) / _S) * bin_w

    valid = (yy >= -1.0) & (yy <= float(H)) & (xx >= -1.0) & (xx <= float(W))
    yc = jnp.where(yy <= 0.0, 0.0, yy)
    xc = jnp.where(xx <= 0.0, 0.0, xx)
    cond_y = jnp.floor(yc) >= (H - 1)
    cond_x = jnp.floor(xc) >= (W - 1)
    y_low = jnp.where(cond_y, H - 1, jnp.floor(yc)).astype(jnp.int32)
    x_low = jnp.where(cond_x, W - 1, jnp.floor(xc)).astype(jnp.int32)
    y_high = jnp.where(cond_y, H - 1, y_low + 1)
    x_high = jnp.where(cond_x, W - 1, x_low + 1)
    ycc = jnp.where(cond_y, float(H - 1), yc)
    xcc = jnp.where(cond_x, float(W - 1), xc)
    ly = ycc - y_low.astype(jnp.float32)
    lx = xcc - x_low.astype(jnp.float32)
    hy = 1.0 - ly
    hx = 1.0 - lx

    wy = jnp.where(cy == 0, hy, ly)
    wx = jnp.where(cx == 0, hx, lx)
    w_ref[...] = jnp.where(valid, wy * wx, 0.0) * (1.0 / (_S * _S))

    ysel = jnp.where(cy == 0, y_low, y_high)
    xsel = jnp.where(cx == 0, x_low, x_high)
    idx_ref[...] = b_ref[:, 0:1] * (H * W) + ysel * W + xsel


def _prep(rois_pad, batches_pad, H, W):
    grid = (_NPAD // _BLK,)
    return pl.pallas_call(
        functools.partial(_prep_body, H=H, W=W),
        grid=grid,
        in_specs=[
            pl.BlockSpec((_BLK, 4), lambda i: (i, 0)),
            pl.BlockSpec((_BLK, 1), lambda i: (i, 0)),
        ],
        out_specs=[
            pl.BlockSpec((_BLK, _NCHUNK * _NROW), lambda i: (i, 0)),
            pl.BlockSpec((_BLK, _NCHUNK * _NROW), lambda i: (i, 0)),
        ],
        out_shape=[
            jax.ShapeDtypeStruct((_NPAD, _NCHUNK * _NROW), jnp.int32),
            jax.ShapeDtypeStruct((_NPAD, _NCHUNK * _NROW), jnp.float32),
        ],
    )(rois_pad, batches_pad)


def _make_sc_pool(C):
    NC = 2
    R_PER = _NPAD // 32
    mesh = plsc.VectorSubcoreMesh(core_axis_name="c", subcore_axis_name="s")

    @functools.partial(
        pl.kernel,
        out_type=jax.ShapeDtypeStruct(
            (_NPAD * _POOL_H * _POOL_W * C,), jnp.float32),
        mesh=mesh,
        compiler_params=pltpu.CompilerParams(needs_layout_passes=False),
        scratch_types=[
            pltpu.VMEM((R_PER * _NCHUNK, _NROW), jnp.int32),
            pltpu.VMEM((R_PER * _NCHUNK * _NROW,), jnp.float32),
            pltpu.VMEM((_NROW, C // 2), jnp.int32),
            pltpu.VMEM((_NROW, C // 2), jnp.int32),
            pltpu.VMEM((_NROW, C // 2), jnp.int32),
            pltpu.VMEM((_NROW, C // 2), jnp.int32),
            pltpu.VMEM((_POOL_W * C,), jnp.float32),
            pltpu.VMEM((_POOL_W * C,), jnp.float32),
            pltpu.SemaphoreType.DMA,
            pltpu.SemaphoreType.DMA,
            pltpu.SemaphoreType.DMA,
            pltpu.SemaphoreType.DMA,
            pltpu.SemaphoreType.DMA,
            pltpu.SemaphoreType.DMA,
        ],
    )
    def sc_pool(featT, idx_hbm, w_hbm, out_hbm,
                idx_v, w_v, rows0, rows1, rows2, rows3, ob0, ob1,
                sem0, sem1, sem2, sem3, osem0, osem1):
        wid = lax.axis_index("s") * NC + lax.axis_index("c")
        bufs = (rows0, rows1, rows2, rows3)
        sems = (sem0, sem1, sem2, sem3)
        obufs = (ob0, ob1)
        osems = (osem0, osem1)
        nch = R_PER * _NCHUNK

        pltpu.sync_copy(idx_hbm.at[wid], idx_v)
        pltpu.sync_copy(w_hbm.at[wid], w_v)
        for k in range(3):
            pltpu.async_copy(featT.at[idx_v.at[k]], bufs[k], sems[k])

        def c4_body(c4, carry):
            for u in range(4):
                c = 4 * c4 + u

                @pl.when(c + 3 < nch)
                def _(u=u):
                    pltpu.async_copy(
                        featT.at[idx_v.at[c + 3]],
                        bufs[(u + 3) % 4], sems[(u + 3) % 4])

                pltpu.make_async_copy(
                    featT.at[idx_v.at[c]], bufs[u], sems[u]).wait()
                rows = bufs[u]
                ph = c % _NCHUNK
                roi = wid * R_PER + c // _NCHUNK
                obase = (roi * _POOL_H * _POOL_W + ph * _POOL_W) * C
                odst = out_hbm.at[pl.ds(obase, _POOL_W * C)]

                ou = u % 2

                @pl.when(c > 1)
                def _(odst=odst, ou=ou):
                    pltpu.make_async_copy(obufs[ou], odst, osems[ou]).wait()

                def pw_body(pw, _, *, c=c, rows=rows, ou=ou):
                    base = pw * _ROWS_PER_BIN
                    wbase = c * _NROW + base
                    wb = [
                        plsc.load_gather(
                            w_v, [jnp.broadcast_to(wbase + jr, (16,))])
                        for jr in range(_ROWS_PER_BIN)
                    ]
                    for cb2 in range(C // 32):
                        raw = plsc.bitcast(
                            rows[base, pl.ds(cb2 * 16, 16)], jnp.bfloat16)
                        v0, v1 = plsc.unpack(
                            raw, format=plsc.PackFormat.INTERLEAVED)
                        acc0 = wb[0] * v0
                        acc1 = wb[0] * v1
                        for jr in range(1, _ROWS_PER_BIN):
                            raw = plsc.bitcast(
                                rows[base + jr, pl.ds(cb2 * 16, 16)],
                                jnp.bfloat16)
                            v0, v1 = plsc.unpack(
                                raw, format=plsc.PackFormat.INTERLEAVED)
                            acc0 = acc0 + wb[jr] * v0
                            acc1 = acc1 + wb[jr] * v1
                        obufs[ou][pl.ds(pw * C + cb2 * 32, 16)] = acc0
                        obufs[ou][pl.ds(pw * C + cb2 * 32 + 16, 16)] = acc1
                    return 0

                lax.fori_loop(0, _POOL_W, pw_body, 0)
                pltpu.async_copy(obufs[ou], odst, osems[ou])

            return carry

        lax.fori_loop(0, nch // 4, c4_body, 0)

        last = (wid * R_PER + R_PER - 1) * _POOL_H * _POOL_W * C
        pltpu.make_async_copy(
            obufs[0],
            out_hbm.at[pl.ds(last + (_NCHUNK - 2) * _POOL_W * C,
                             _POOL_W * C)],
            osems[0]).wait()
        pltpu.make_async_copy(
            obufs[1],
            out_hbm.at[pl.ds(last + (_NCHUNK - 1) * _POOL_W * C,
                             _POOL_W * C)],
            osems[1]).wait()

    return sc_pool


def _transpose_body(in_ref, out_ref):
    out_ref[...] = jnp.transpose(in_ref[...], (0, 2, 1))


def _to_channel_major(out3, N, C):
    blk = 8
    return pl.pallas_call(
        _transpose_body,
        grid=(N // blk,),
        in_specs=[pl.BlockSpec((blk, _POOL_H * _POOL_W, C),
                               lambda i: (i, 0, 0))],
        out_specs=pl.BlockSpec((blk, C, _POOL_H * _POOL_W),
                               lambda i: (i, 0, 0)),
        out_shape=jax.ShapeDtypeStruct((N, C, _POOL_H * _POOL_W),
                                       jnp.float32),
    )(out3)


def kernel(feat, rois, roibatches):
    B, C, H, W = feat.shape
    N = rois.shape[0]

    perm = np.arange(C).reshape(C // 32, 2, 16).transpose(0, 2, 1).reshape(C)
    featT = jnp.transpose(feat[:, perm], (0, 2, 3, 1))
    featT = featT.astype(jnp.bfloat16).reshape(B * H * W, C // 2, 2)
    featT = lax.bitcast_convert_type(featT, jnp.int32)

    rois_pad = jnp.pad(rois, ((0, _NPAD - N), (0, 0)))
    batches_pad = jnp.pad(roibatches, (0, _NPAD - N)).reshape(_NPAD, 1)

    idx, w = _prep(rois_pad, batches_pad, H, W)
    r_per = _NPAD // 32
    idx = idx.reshape(32, r_per * _NCHUNK, _NROW)
    w = w.reshape(32, r_per * _NCHUNK * _NROW)

    out = _make_sc_pool(C)(featT, idx, w)
    out = out.reshape(_NPAD, _POOL_H * _POOL_W, C)
    out = _to_channel_major(out, N, C)
    return out.reshape(N, C, _POOL_H, _POOL_W)


# speedup vs baseline: 1.2699x; 1.2699x over previous
import functools

import jax
import jax.numpy as jnp
import numpy as np
from jax import lax
from jax.experimental import pallas as pl
from jax.experimental.pallas import tpu as pltpu
from jax.experimental.pallas import tpu_sc as plsc

_POOL_H = 7
_POOL_W = 7
_SCALE = 0.125
_S = 2

_NPAD = 1024
_NROW = 112
_NCHUNK = 7
_ROWS_PER_BIN = 16
_BLK = 128


def _prep_body(rois_ref, b_ref, idx_ref, w_ref, *, H, W):
    x1 = rois_ref[:, 0:1] * _SCALE
    y1 = rois_ref[:, 1:2] * _SCALE
    x2 = rois_ref[:, 2:3] * _SCALE
    y2 = rois_ref[:, 3:4] * _SCALE
    roi_w = jnp.maximum(x2 - x1, 1.0)
    roi_h = jnp.maximum(y2 - y1, 1.0)
    bin_w = roi_w / _POOL_W
    bin_h = roi_h / _POOL_H

    j = lax.broadcasted_iota(jnp.int32, (_BLK, _NCHUNK * _NROW), 1)
    ph = j // _NROW
    pw = (j % _NROW) // _ROWS_PER_BIN
    s = (j % _ROWS_PER_BIN) // 4
    iy = s // 2
    ix = s % 2
    c = j % 4
    cy = c // 2
    cx = c % 2

    yy = y1 + (ph.astype(jnp.float32)
               + (iy.astype(jnp.float32) + 0.5) / _S) * bin_h
    xx = x1 + (pw.astype(jnp.float32)
               + (ix.astype(jnp.float32) + 0.5) / _S) * bin_w

    valid = (yy >= -1.0) & (yy <= float(H)) & (xx >= -1.0) & (xx <= float(W))
    yc = jnp.where(yy <= 0.0, 0.0, yy)
    xc = jnp.where(xx <= 0.0, 0.0, xx)
    cond_y = jnp.floor(yc) >= (H - 1)
    cond_x = jnp.floor(xc) >= (W - 1)
    y_low = jnp.where(cond_y, H - 1, jnp.floor(yc)).astype(jnp.int32)
    x_low = jnp.where(cond_x, W - 1, jnp.floor(xc)).astype(jnp.int32)
    y_high = jnp.where(cond_y, H - 1, y_low + 1)
    x_high = jnp.where(cond_x, W - 1, x_low + 1)
    ycc = jnp.where(cond_y, float(H - 1), yc)
    xcc = jnp.where(cond_x, float(W - 1), xc)
    ly = ycc - y_low.astype(jnp.float32)
    lx = xcc - x_low.astype(jnp.float32)
    hy = 1.0 - ly
    hx = 1.0 - lx

    wy = jnp.where(cy == 0, hy, ly)
    wx = jnp.where(cx == 0, hx, lx)
    w_ref[...] = jnp.where(valid, wy * wx, 0.0) * (1.0 / (_S * _S))

    ysel = jnp.where(cy == 0, y_low, y_high)
    xsel = jnp.where(cx == 0, x_low, x_high)
    idx_ref[...] = b_ref[:, 0:1] * (H * W) + ysel * W + xsel


def _prep(rois_pad, batches_pad, H, W):
    grid = (_NPAD // _BLK,)
    return pl.pallas_call(
        functools.partial(_prep_body, H=H, W=W),
        grid=grid,
        in_specs=[
            pl.BlockSpec((_BLK, 4), lambda i: (i, 0)),
            pl.BlockSpec((_BLK, 1), lambda i: (i, 0)),
        ],
        out_specs=[
            pl.BlockSpec((_BLK, _NCHUNK * _NROW), lambda i: (i, 0)),
            pl.BlockSpec((_BLK, _NCHUNK * _NROW), lambda i: (i, 0)),
        ],
        out_shape=[
            jax.ShapeDtypeStruct((_NPAD, _NCHUNK * _NROW), jnp.int32),
            jax.ShapeDtypeStruct((_NPAD, _NCHUNK * _NROW), jnp.float32),
        ],
    )(rois_pad, batches_pad)


def _make_sc_pool(C):
    NC = 2
    R_PER = _NPAD // 32
    mesh = plsc.VectorSubcoreMesh(core_axis_name="c", subcore_axis_name="s")

    @functools.partial(
        pl.kernel,
        out_type=jax.ShapeDtypeStruct(
            (_NPAD * _POOL_H * _POOL_W * C,), jnp.float32),
        mesh=mesh,
        compiler_params=pltpu.CompilerParams(needs_layout_passes=False),
        scratch_types=[
            pltpu.VMEM((R_PER * _NCHUNK, _NROW), jnp.int32),
            pltpu.VMEM((R_PER * _NCHUNK * _NROW,), jnp.float32),
            pltpu.VMEM((_NROW, C // 2), jnp.int32),
            pltpu.VMEM((_NROW, C // 2), jnp.int32),
            pltpu.VMEM((_NROW, C // 2), jnp.int32),
            pltpu.VMEM((_NROW, C // 2), jnp.int32),
            pltpu.VMEM((_POOL_W * C,), jnp.float32),
            pltpu.VMEM((_POOL_W * C,), jnp.float32),
            pltpu.SemaphoreType.DMA,
            pltpu.SemaphoreType.DMA,
            pltpu.SemaphoreType.DMA,
            pltpu.SemaphoreType.DMA,
            pltpu.SemaphoreType.DMA,
            pltpu.SemaphoreType.DMA,
        ],
    )
    def sc_pool(featT, idx_hbm, w_hbm, out_hbm,
                idx_v, w_v, rows0, rows1, rows2, rows3, ob0, ob1,
                sem0, sem1, sem2, sem3, osem0, osem1):
        wid = lax.axis_index("s") * NC + lax.axis_index("c")
        bufs = (rows0, rows1, rows2, rows3)
        sems = (sem0, sem1, sem2, sem3)
        obufs = (ob0, ob1)
        osems = (osem0, osem1)
        nch = R_PER * _NCHUNK

        pltpu.sync_copy(idx_hbm.at[wid], idx_v)
        pltpu.sync_copy(w_hbm.at[wid], w_v)
        for k in range(3):
            pltpu.async_copy(featT.at[idx_v.at[k]], bufs[k], sems[k])

        def c4_body(c4, carry):
            for u in range(4):
                c = 4 * c4 + u

                @pl.when(c + 3 < nch)
                def _(u=u):
                    pltpu.async_copy(
                        featT.at[idx_v.at[c + 3]],
                        bufs[(u + 3) % 4], sems[(u + 3) % 4])

                pltpu.make_async_copy(
                    featT.at[idx_v.at[c]], bufs[u], sems[u]).wait()
                rows = bufs[u]
                ph = c % _NCHUNK
                roi = wid * R_PER + c // _NCHUNK
                obase = (roi * _POOL_H * _POOL_W + ph * _POOL_W) * C
                odst = out_hbm.at[pl.ds(obase, _POOL_W * C)]

                ou = u % 2

                @pl.when(c > 1)
                def _(odst=odst, ou=ou):
                    pltpu.make_async_copy(obufs[ou], odst, osems[ou]).wait()

                def pw_body(pw, _, *, c=c, rows=rows, ou=ou):
                    base = pw * _ROWS_PER_BIN
                    wbase = c * _NROW + base
                    wb = [
                        plsc.load_gather(
                            w_v, [jnp.broadcast_to(wbase + jr, (16,))])
                        for jr in range(_ROWS_PER_BIN)
                    ]
                    for cb2 in range(C // 32):
                        raw = plsc.bitcast(
                            rows[base, pl.ds(cb2 * 16, 16)], jnp.bfloat16)
                        v0, v1 = plsc.unpack(
                            raw, format=plsc.PackFormat.INTERLEAVED)
                        acc0 = wb[0] * v0
                        acc1 = wb[0] * v1
                        for jr in range(1, _ROWS_PER_BIN):
                            raw = plsc.bitcast(
                                rows[base + jr, pl.ds(cb2 * 16, 16)],
                                jnp.bfloat16)
                            v0, v1 = plsc.unpack(
                                raw, format=plsc.PackFormat.INTERLEAVED)
                            acc0 = acc0 + wb[jr] * v0
                            acc1 = acc1 + wb[jr] * v1
                        obufs[ou][pl.ds(pw * C + cb2 * 32, 16)] = acc0
                        obufs[ou][pl.ds(pw * C + cb2 * 32 + 16, 16)] = acc1
                    return 0

                lax.fori_loop(0, _POOL_W, pw_body, 0)
                pltpu.async_copy(obufs[ou], odst, osems[ou])

            return carry

        lax.fori_loop(0, nch // 4, c4_body, 0)

        last = (wid * R_PER + R_PER - 1) * _POOL_H * _POOL_W * C
        pltpu.make_async_copy(
            obufs[0],
            out_hbm.at[pl.ds(last + (_NCHUNK - 2) * _POOL_W * C,
                             _POOL_W * C)],
            osems[0]).wait()
        pltpu.make_async_copy(
            obufs[1],
            out_hbm.at[pl.ds(last + (_NCHUNK - 1) * _POOL_W * C,
                             _POOL_W * C)],
            osems[1]).wait()

    return sc_pool


def _transpose_body(in_ref, out_ref):
    out_ref[...] = jnp.transpose(in_ref[...], (0, 2, 1))


def _to_channel_major(out3, N, C):
    blk = 8
    return pl.pallas_call(
        _transpose_body,
        grid=(N // blk,),
        in_specs=[pl.BlockSpec((blk, _POOL_H * _POOL_W, C),
                               lambda i: (i, 0, 0))],
        out_specs=pl.BlockSpec((blk, C, _POOL_H * _POOL_W),
                               lambda i: (i, 0, 0)),
        out_shape=jax.ShapeDtypeStruct((N, C, _POOL_H * _POOL_W),
                                       jnp.float32),
    )(out3)


def kernel(feat, rois, roibatches):
    B, C, H, W = feat.shape
    N = rois.shape[0]

    featT = jnp.transpose(feat, (0, 2, 3, 1)).reshape(B * H * W, C)
    featT = (featT.reshape(B * H * W, C // 32, 2, 16)
             .transpose(0, 1, 3, 2).reshape(B * H * W, C // 2, 2)
             .astype(jnp.bfloat16))
    featT = lax.bitcast_convert_type(featT, jnp.int32)

    rois_pad = jnp.pad(rois, ((0, _NPAD - N), (0, 0)))
    batches_pad = jnp.pad(roibatches, (0, _NPAD - N)).reshape(_NPAD, 1)

    idx, w = _prep(rois_pad, batches_pad, H, W)
    r_per = _NPAD // 32
    idx = idx.reshape(32, r_per * _NCHUNK, _NROW)
    w = w.reshape(32, r_per * _NCHUNK * _NROW)

    out = _make_sc_pool(C)(featT, idx, w)
    out = out.reshape(_NPAD, _POOL_H * _POOL_W, C)
    out = _to_channel_major(out, N, C)
    return out.reshape(N, C, _POOL_H, _POOL_W)
